# baseline (speedup 1.0000x reference)
import functools

import jax
import jax.numpy as jnp
from jax import lax
from jax.experimental import pallas as pl
from jax.experimental.pallas import tpu as pltpu
from jax.experimental.pallas import tpu_sc as plsc

VOCAB = 1000000
D = 32
S = 50
B = 4096
N = B * S
NC, NS = 2, 16
NW = NC * NS
PER_W = N // NW
CH = 800
NCHUNK = PER_W // CH

_mesh = plsc.VectorSubcoreMesh(core_axis_name="c", subcore_axis_name="s")


@functools.partial(
    pl.kernel,
    mesh=_mesh,
    out_type=jax.ShapeDtypeStruct((N, 128), jnp.float32),
    scratch_types=[
        pltpu.VMEM((PER_W,), jnp.int32),
        pltpu.VMEM((CH, 128), jnp.float32),
        pltpu.SemaphoreType.DMA,
    ],
)
def _sc_gather(tp_hbm, idx_hbm, out_hbm, idx_v, rows_v, sem):
    wid = lax.axis_index("s") * NC + lax.axis_index("c")
    base = wid * PER_W
    pltpu.sync_copy(idx_hbm.at[pl.ds(base, PER_W)], idx_v)
    for i in range(NCHUNK):
        off = i * CH
        pltpu.async_copy(
            tp_hbm.at[idx_v.at[pl.ds(off, CH)]], rows_v, sem
        ).wait()
        pltpu.sync_copy(rows_v, out_hbm.at[pl.ds(base + off, CH)])


def _mm_body(g_ref, w_ref, b_ref, o_ref):
    s = pl.program_id(0)

    @pl.when(s == 0)
    def _():
        o_ref[...] = jnp.broadcast_to(b_ref[...], (B, D))

    o_ref[...] += lax.dot_general(
        g_ref[...], w_ref[0],
        (((1,), (0,)), ((), ())),
        preferred_element_type=jnp.float32,
    )


def _tc_matmul(g, w4, b):
    return pl.pallas_call(
        _mm_body,
        grid=(S,),
        in_specs=[
            pl.BlockSpec((B, 128), lambda s: (s, 0)),
            pl.BlockSpec((1, 128, D), lambda s: (s, 0, 0)),
            pl.BlockSpec((1, D), lambda s: (0, 0)),
        ],
        out_specs=pl.BlockSpec((B, D), lambda s: (0, 0)),
        out_shape=jax.ShapeDtypeStruct((B, D), jnp.float32),
    )(g, w4, b)


def kernel(x, table, W, b):
    xi = x.astype(jnp.int32)
    idx = xi.T.reshape(N)
    tp = jnp.pad(table, ((0, 0), (0, 96)))
    g = _sc_gather(tp, idx)
    w4 = jnp.tile(W.T.reshape(S, D, D), (1, 4, 1))
    return _tc_matmul(g, w4, b.reshape(1, D))


# speedup vs baseline: 1.1060x; 1.1060x over previous
import functools

import jax
import jax.numpy as jnp
from jax import lax
from jax.experimental import pallas as pl
from jax.experimental.pallas import tpu as pltpu
from jax.experimental.pallas import tpu_sc as plsc

VOCAB = 1000000
D = 32
S = 50
B = 4096
N = B * S
NC, NS = 2, 16
NW = NC * NS
PER_W = N // NW
CH = 1600
NCHUNK = PER_W // CH

_mesh = plsc.VectorSubcoreMesh(core_axis_name="c", subcore_axis_name="s")


@functools.partial(
    pl.kernel,
    mesh=_mesh,
    out_type=jax.ShapeDtypeStruct((N, D), jnp.float32),
    scratch_types=[
        pltpu.VMEM((PER_W,), jnp.int32),
        pltpu.VMEM((CH, D), jnp.float32),
        pltpu.SemaphoreType.DMA,
    ],
    compiler_params=pltpu.CompilerParams(use_tc_tiling_on_sc=False),
)
def _sc_gather(table_hbm, idx_hbm, out_hbm, idx_v, rows_v, sem):
    wid = lax.axis_index("s") * NC + lax.axis_index("c")
    base = wid * PER_W
    pltpu.sync_copy(idx_hbm.at[pl.ds(base, PER_W)], idx_v)
    for i in range(NCHUNK):
        off = i * CH
        pltpu.async_copy(
            table_hbm.at[idx_v.at[pl.ds(off, CH)]], rows_v, sem
        ).wait()
        pltpu.sync_copy(rows_v, out_hbm.at[pl.ds(base + off, CH)])


def _mm_body(g_ref, w_ref, b_ref, o_ref):
    o_ref[...] = (
        lax.dot_general(
            g_ref[...], w_ref[...],
            (((1,), (1,)), ((), ())),
            preferred_element_type=jnp.float32,
        )
        + b_ref[...]
    )


_BB = 512


def _tc_matmul(g, w, b):
    return pl.pallas_call(
        _mm_body,
        grid=(B // _BB,),
        in_specs=[
            pl.BlockSpec((_BB, S * D), lambda i: (i, 0)),
            pl.BlockSpec((D, S * D), lambda i: (0, 0)),
            pl.BlockSpec((1, D), lambda i: (0, 0)),
        ],
        out_specs=pl.BlockSpec((_BB, D), lambda i: (i, 0)),
        out_shape=jax.ShapeDtypeStruct((B, D), jnp.float32),
    )(g, w, b)


def kernel(x, table, W, b):
    xf = x.reshape(N).astype(jnp.int32)
    gathered = _sc_gather(table, xf)
    return _tc_matmul(gathered.reshape(B, S * D), W, b.reshape(1, D))
